# baseline (speedup 1.0000x reference)
import functools

import numpy as np
import jax
import jax.numpy as jnp
from jax import lax
from jax.experimental import pallas as pl
from jax.experimental.pallas import tpu as pltpu
from jax.experimental.pallas import tpu_sc as plsc

IN_CH = 256
N_HEADS = 4
HEAD_DIM = IN_CH // N_HEADS
N_NODES = 10000
N_EDGES = 160000

_NC = 2
_NS = 16
_NW = _NC * _NS
_L = 16

_EPW = N_EDGES // _NW
_C = 40
_NCHUNK = _EPW // _C


_EXPAND = np.repeat(np.eye(N_HEADS, dtype=np.float32), HEAD_DIM, axis=1)
_E_SRC = np.concatenate([_EXPAND, np.zeros_like(_EXPAND)], axis=0)
_E_TGT = np.concatenate([np.zeros_like(_EXPAND), _EXPAND], axis=0)


def _node_tables_body(x_ref, wst_ref, es_ref, et_ref, wp_ref, b_ref,
                      u_ref, v_ref):
    x = x_ref[...]
    g = jax.nn.sigmoid(
        jnp.dot(x, wst_ref[...], preferred_element_type=jnp.float32))
    gs = jnp.dot(g, es_ref[...], preferred_element_type=jnp.float32)
    gt = jnp.dot(g, et_ref[...], preferred_element_type=jnp.float32)
    wp = wp_ref[...]
    u_ref[...] = jnp.dot(gs * x, wp, preferred_element_type=jnp.float32) + b_ref[...]
    v_ref[...] = jnp.dot(gt * x, wp, preferred_element_type=jnp.float32)


def _node_tables(x, w_st, e_src, e_tgt, w_proj, b2):
    blk = 1000
    grid = (N_NODES // blk,)
    return pl.pallas_call(
        _node_tables_body,
        grid=grid,
        in_specs=[
            pl.BlockSpec((blk, IN_CH), lambda i: (i, 0)),
            pl.BlockSpec((IN_CH, 8), lambda i: (0, 0)),
            pl.BlockSpec((8, IN_CH), lambda i: (0, 0)),
            pl.BlockSpec((8, IN_CH), lambda i: (0, 0)),
            pl.BlockSpec((IN_CH, IN_CH), lambda i: (0, 0)),
            pl.BlockSpec((1, IN_CH), lambda i: (0, 0)),
        ],
        out_specs=[
            pl.BlockSpec((blk, IN_CH), lambda i: (i, 0)),
            pl.BlockSpec((blk, IN_CH), lambda i: (i, 0)),
        ],
        out_shape=[jax.ShapeDtypeStruct((N_NODES, IN_CH), jnp.float32),
                   jax.ShapeDtypeStruct((N_NODES, IN_CH), jnp.float32)],
    )(x, w_st, e_src, e_tgt, w_proj, b2)


_mesh = plsc.VectorSubcoreMesh(core_axis_name="c", subcore_axis_name="s")


_NSLOT = 4
_RU = 8


@functools.partial(
    pl.kernel,
    mesh=_mesh,
    out_type=jax.ShapeDtypeStruct((N_EDGES, IN_CH), jnp.float32),
    scratch_types=(
        [pltpu.VMEM((_EPW,), jnp.int32),
         pltpu.VMEM((_EPW,), jnp.int32)]
        + [pltpu.VMEM((_C, IN_CH), jnp.float32)] * (2 * _NSLOT)
        + [pltpu.SemaphoreType.DMA] * (2 * _NSLOT)
    ),
)
def _edge_combine(u_hbm, v_hbm, src_hbm, tgt_hbm, out_hbm,
                  idx_s, idx_t, *bufs):
    rus = bufs[0:2 * _NSLOT:2]
    rvs = bufs[1:2 * _NSLOT:2]
    sgs = bufs[2 * _NSLOT:3 * _NSLOT]
    sws = bufs[3 * _NSLOT:4 * _NSLOT]

    wid = lax.axis_index("s") * _NC + lax.axis_index("c")
    base = wid * _EPW
    pltpu.sync_copy(src_hbm.at[pl.ds(base, _EPW)], idx_s)
    pltpu.sync_copy(tgt_hbm.at[pl.ds(base, _EPW)], idx_t)

    def fire(j, p):
        off = j * _C
        pltpu.async_copy(u_hbm.at[idx_s.at[pl.ds(off, _C)]], rus[p], sgs[p])
        pltpu.async_copy(v_hbm.at[idx_t.at[pl.ds(off, _C)]], rvs[p], sgs[p])

    def drain_gather(p):
        pltpu.make_async_copy(u_hbm.at[pl.ds(0, _C)], rus[p], sgs[p]).wait()
        pltpu.make_async_copy(v_hbm.at[pl.ds(0, _C)], rvs[p], sgs[p]).wait()

    def drain_wo(p):
        pltpu.make_async_copy(rus[p], out_hbm.at[pl.ds(0, _C)], sws[p]).wait()

    def add_and_fire_wo(j, p):
        def add_rows(i, c):
            i0 = i * _RU
            for r in range(_RU):
                for k in range(IN_CH // _L):
                    sl = pl.ds(k * _L, _L)
                    plsc.addupdate(rus[p].at[i0 + r, sl], rvs[p][i0 + r, sl])
            return c
        lax.fori_loop(0, _C // _RU, add_rows, 0)
        pltpu.async_copy(rus[p], out_hbm.at[pl.ds(base + j * _C, _C)], sws[p])

    fire(0, 0)
    fire(1, 1)

    fire(2, 2)
    drain_gather(0)
    add_and_fire_wo(0, 0)
    fire(3, 3)
    drain_gather(1)
    add_and_fire_wo(1, 1)
    drain_wo(0)
    fire(4, 0)
    drain_gather(2)
    add_and_fire_wo(2, 2)
    drain_wo(1)
    fire(5, 1)
    drain_gather(3)
    add_and_fire_wo(3, 3)

    def body(g, carry):
        j0 = 4 * g
        for p in range(_NSLOT):
            j = j0 + p
            pf = (p + 2) % _NSLOT

            @pl.when(j + 2 < _NCHUNK)
            def _():
                drain_wo(pf)
                fire(j + 2, pf)
            drain_gather(p)
            add_and_fire_wo(j, p)
        return carry

    lax.fori_loop(1, (_NCHUNK - 1) // _NSLOT, body, 0)

    drain_gather(0)
    add_and_fire_wo(_NCHUNK - 1, 0)
    drain_wo(1)
    drain_wo(2)
    drain_wo(3)
    drain_wo(0)


def kernel(node_embeddings, edge_index, W_src, W_tgt, W_proj, b_proj):
    eidx = edge_index.astype(jnp.int32)
    w_st = jnp.concatenate([W_src, W_tgt], axis=1)
    u_tab, v_tab = _node_tables(node_embeddings, w_st, _E_SRC, _E_TGT,
                                W_proj, b_proj.reshape(1, IN_CH))
    return _edge_combine(u_tab, v_tab, eidx[0], eidx[1])


# speedup vs baseline: 1.1889x; 1.1889x over previous
import functools

import numpy as np
import jax
import jax.numpy as jnp
from jax import lax
from jax.experimental import pallas as pl
from jax.experimental.pallas import tpu as pltpu
from jax.experimental.pallas import tpu_sc as plsc

IN_CH = 256
N_HEADS = 4
HEAD_DIM = IN_CH // N_HEADS
N_NODES = 10000
N_EDGES = 160000

_NC = 2
_NS = 16
_NW = _NC * _NS
_L = 16

_EPW = N_EDGES // _NW
_C = 40
_NCHUNK = _EPW // _C


_EXPAND = np.repeat(np.eye(N_HEADS, dtype=np.float32), HEAD_DIM, axis=1)
_E_SRC = np.concatenate([_EXPAND, np.zeros_like(_EXPAND)], axis=0)
_E_TGT = np.concatenate([np.zeros_like(_EXPAND), _EXPAND], axis=0)


def _node_tables_body(x_ref, wst_ref, es_ref, et_ref, wp_ref, b_ref,
                      u_ref, v_ref):
    x = x_ref[...]
    g = jax.nn.sigmoid(
        jnp.dot(x, wst_ref[...], preferred_element_type=jnp.float32))
    gs = jnp.dot(g, es_ref[...], preferred_element_type=jnp.float32)
    gt = jnp.dot(g, et_ref[...], preferred_element_type=jnp.float32)
    wp = wp_ref[...]
    u_ref[...] = jnp.dot(gs * x, wp, preferred_element_type=jnp.float32) + b_ref[...]
    v_ref[...] = jnp.dot(gt * x, wp, preferred_element_type=jnp.float32)


def _node_tables(x, w_st, e_src, e_tgt, w_proj, b2):
    blk = 2000
    grid = (N_NODES // blk,)
    return pl.pallas_call(
        _node_tables_body,
        grid=grid,
        in_specs=[
            pl.BlockSpec((blk, IN_CH), lambda i: (i, 0)),
            pl.BlockSpec((IN_CH, 8), lambda i: (0, 0)),
            pl.BlockSpec((8, IN_CH), lambda i: (0, 0)),
            pl.BlockSpec((8, IN_CH), lambda i: (0, 0)),
            pl.BlockSpec((IN_CH, IN_CH), lambda i: (0, 0)),
            pl.BlockSpec((1, IN_CH), lambda i: (0, 0)),
        ],
        out_specs=[
            pl.BlockSpec((blk, IN_CH), lambda i: (i, 0)),
            pl.BlockSpec((blk, IN_CH), lambda i: (i, 0)),
        ],
        out_shape=[jax.ShapeDtypeStruct((N_NODES, IN_CH), jnp.float32),
                   jax.ShapeDtypeStruct((N_NODES, IN_CH), jnp.float32)],
    )(x, w_st, e_src, e_tgt, w_proj, b2)


_mesh = plsc.VectorSubcoreMesh(core_axis_name="c", subcore_axis_name="s")


_NSLOT = 4


@functools.partial(
    pl.kernel,
    mesh=_mesh,
    out_type=jax.ShapeDtypeStruct((N_EDGES, IN_CH), jnp.float32),
    scratch_types=(
        [pltpu.VMEM((_EPW,), jnp.int32),
         pltpu.VMEM((_EPW,), jnp.int32)]
        + [pltpu.VMEM((_C, IN_CH), jnp.float32)] * (2 * _NSLOT)
        + [pltpu.SemaphoreType.DMA] * (2 * _NSLOT)
    ),
)
def _edge_combine(u_hbm, v_hbm, src_hbm, tgt_hbm, out_hbm,
                  idx_s, idx_t, *bufs):
    rus = bufs[0:2 * _NSLOT:2]
    rvs = bufs[1:2 * _NSLOT:2]
    sgs = bufs[2 * _NSLOT:3 * _NSLOT]
    sws = bufs[3 * _NSLOT:4 * _NSLOT]

    wid = lax.axis_index("s") * _NC + lax.axis_index("c")
    base = wid * _EPW
    pltpu.sync_copy(src_hbm.at[pl.ds(base, _EPW)], idx_s)
    pltpu.sync_copy(tgt_hbm.at[pl.ds(base, _EPW)], idx_t)

    def fire(j, p):
        off = j * _C
        pltpu.async_copy(u_hbm.at[idx_s.at[pl.ds(off, _C)]], rus[p], sgs[p])
        pltpu.async_copy(v_hbm.at[idx_t.at[pl.ds(off, _C)]], rvs[p], sgs[p])

    def drain_gather(p):
        pltpu.make_async_copy(u_hbm.at[pl.ds(0, _C)], rus[p], sgs[p]).wait()
        pltpu.make_async_copy(v_hbm.at[pl.ds(0, _C)], rvs[p], sgs[p]).wait()

    def drain_wo(p):
        pltpu.make_async_copy(rus[p], out_hbm.at[pl.ds(0, _C)], sws[p]).wait()

    def add_and_fire_wo(j, p):
        def add_row(i, c):
            for k in range(IN_CH // _L):
                sl = pl.ds(k * _L, _L)
                plsc.addupdate(rus[p].at[i, sl], rvs[p][i, sl])
            return c
        lax.fori_loop(0, _C, add_row, 0)
        pltpu.async_copy(rus[p], out_hbm.at[pl.ds(base + j * _C, _C)], sws[p])

    fire(0, 0)
    fire(1, 1)

    fire(2, 2)
    drain_gather(0)
    add_and_fire_wo(0, 0)
    fire(3, 3)
    drain_gather(1)
    add_and_fire_wo(1, 1)
    drain_wo(0)
    fire(4, 0)
    drain_gather(2)
    add_and_fire_wo(2, 2)
    drain_wo(1)
    fire(5, 1)
    drain_gather(3)
    add_and_fire_wo(3, 3)

    def body(g, carry):
        j0 = 4 * g
        for p in range(_NSLOT):
            j = j0 + p
            pf = (p + 2) % _NSLOT

            @pl.when(j + 2 < _NCHUNK)
            def _():
                drain_wo(pf)
                fire(j + 2, pf)
            drain_gather(p)
            add_and_fire_wo(j, p)
        return carry

    lax.fori_loop(1, (_NCHUNK - 1) // _NSLOT, body, 0)

    drain_gather(0)
    add_and_fire_wo(_NCHUNK - 1, 0)
    drain_wo(1)
    drain_wo(2)
    drain_wo(3)
    drain_wo(0)


def kernel(node_embeddings, edge_index, W_src, W_tgt, W_proj, b_proj):
    eidx = edge_index.astype(jnp.int32)
    w_st = jnp.concatenate([W_src, W_tgt], axis=1)
    u_tab, v_tab = _node_tables(node_embeddings, w_st, _E_SRC, _E_TGT,
                                W_proj, b_proj.reshape(1, IN_CH))
    return _edge_combine(u_tab, v_tab, eidx[0], eidx[1])
